# Initial kernel scaffold; baseline (speedup 1.0000x reference)
import jax
import jax.numpy as jnp
from jax.experimental import pallas as pl


def kernel(hidden_states, gate_w, gate_up_w, down_w):
    raise NotImplementedError("write your pallas kernel here")


# speedup vs baseline: 1.0064x; 1.0064x over previous
import functools

import jax
import jax.numpy as jnp
from jax.experimental import pallas as pl
from jax.experimental.pallas import tpu as pltpu

NUM_EXPERTS = 8
TOP_K = 2
D_MODEL = 2048
D_FF = 1408
T_TOKENS = 8192

ROUTER_TB = 1024
MLP_TB = 512


def _router_body(x_ref, gw_ref, wd_ref, xbf_ref):
    x = x_ref[...]
    logits = jax.lax.dot_general(
        x, gw_ref[...], (((1,), (0,)), ((), ())),
        preferred_element_type=jnp.float32)
    p = jax.nn.softmax(logits, axis=-1)
    m1 = jnp.max(p, axis=-1, keepdims=True)
    p_wo = jnp.where(p >= m1, -jnp.inf, p)
    m2 = jnp.max(p_wo, axis=-1, keepdims=True)
    mask = p >= m2
    wd = jnp.where(mask, p, 0.0) / (m1 + m2)
    wd_ref[...] = wd
    xbf_ref[...] = x.astype(jnp.bfloat16)


def _mlp_body(xbf_ref, gup_ref, dwn_ref, wd_ref, out_ref):
    e = pl.program_id(1)

    @pl.when(e == 0)
    def _():
        out_ref[...] = jnp.zeros_like(out_ref)

    x = xbf_ref[...]
    g = jax.lax.dot_general(
        x, gup_ref[0], (((1,), (0,)), ((), ())),
        preferred_element_type=jnp.float32)
    a = (jax.nn.silu(g[:, :D_FF]) * g[:, D_FF:]).astype(jnp.bfloat16)
    y = jax.lax.dot_general(
        a, dwn_ref[0], (((1,), (0,)), ((), ())),
        preferred_element_type=jnp.float32)
    wd = wd_ref[...]
    lane = jax.lax.broadcasted_iota(jnp.int32, wd.shape, 1)
    w = jnp.sum(jnp.where(lane == e, wd, 0.0), axis=1, keepdims=True)
    out_ref[...] += w * y


def kernel(hidden_states, gate_w, gate_up_w, down_w):
    T, D = hidden_states.shape
    E = NUM_EXPERTS

    wd, xbf = pl.pallas_call(
        _router_body,
        grid=(T // ROUTER_TB,),
        in_specs=[
            pl.BlockSpec((ROUTER_TB, D), lambda t: (t, 0)),
            pl.BlockSpec((D, E), lambda t: (0, 0)),
        ],
        out_specs=[
            pl.BlockSpec((ROUTER_TB, E), lambda t: (t, 0)),
            pl.BlockSpec((ROUTER_TB, D), lambda t: (t, 0)),
        ],
        out_shape=[
            jax.ShapeDtypeStruct((T, E), jnp.float32),
            jax.ShapeDtypeStruct((T, D), jnp.bfloat16),
        ],
    )(hidden_states, gate_w)

    gup_bf = gate_up_w.astype(jnp.bfloat16)
    dwn_bf = down_w.astype(jnp.bfloat16)

    out = pl.pallas_call(
        _mlp_body,
        grid=(T // MLP_TB, E),
        in_specs=[
            pl.BlockSpec((MLP_TB, D), lambda t, e: (t, 0)),
            pl.BlockSpec((1, D, 2 * D_FF), lambda t, e: (e, 0, 0)),
            pl.BlockSpec((1, D_FF, D), lambda t, e: (e, 0, 0)),
            pl.BlockSpec((MLP_TB, E), lambda t, e: (t, 0)),
        ],
        out_specs=pl.BlockSpec((MLP_TB, D), lambda t, e: (t, 0)),
        out_shape=jax.ShapeDtypeStruct((T, D), jnp.float32),
        compiler_params=pltpu.CompilerParams(
            dimension_semantics=("parallel", "arbitrary"),
        ),
    )(xbf, gup_bf, dwn_bf, wd)
    return out
